# baseline (speedup 1.0000x reference)
import jax
import jax.numpy as jnp
from jax.experimental import pallas as pl

_BPB = 2


def _pos_body(t_ref, out_ref):
    d2 = t_ref.shape[0]
    hw = out_ref.shape[1]
    w2 = t_ref.shape[1]
    w = w2 // 2
    k_col = jax.lax.broadcasted_iota(jnp.int32, (w2, hw), 1)
    j_row = jax.lax.broadcasted_iota(jnp.int32, (w2, hw), 0)
    m = (((k_col % w) == j_row) | ((k_col // w + w) == j_row)).astype(
        jnp.float32
    )
    plane = jnp.dot(t_ref[...], m, preferred_element_type=jnp.float32)
    for q in range(out_ref.shape[0] // d2):
        out_ref[q * d2 : (q + 1) * d2, :] = plane


def kernel(x, row_embed, col_embed):
    b = x.shape[0]
    h, w = x.shape[-2], x.shape[-1]
    d = row_embed.shape[1]
    col_t = col_embed[:w].T
    row_t = row_embed[:h].T
    z_cw = jnp.zeros((d, h), col_t.dtype)
    z_rh = jnp.zeros((d, w), row_t.dtype)
    t = jnp.concatenate(
        [
            jnp.concatenate([col_t, z_cw], axis=1),
            jnp.concatenate([z_rh, row_t], axis=1),
        ],
        axis=0,
    )
    out = pl.pallas_call(
        _pos_body,
        grid=(b // _BPB,),
        in_specs=[pl.BlockSpec((2 * d, w + h), lambda i: (0, 0))],
        out_specs=pl.BlockSpec((_BPB * 2 * d, h * w), lambda i: (i, 0)),
        out_shape=jax.ShapeDtypeStruct((b * 2 * d, h * w), x.dtype),
    )(t)
    return out.reshape(b, 2 * d, h, w)
